# Initial kernel scaffold; baseline (speedup 1.0000x reference)
import jax
import jax.numpy as jnp
from jax.experimental import pallas as pl


def kernel(x, edge_index, W, b):
    raise NotImplementedError("write your pallas kernel here")


# speedup vs baseline: 33.0322x; 33.0322x over previous
import functools

import jax
import jax.numpy as jnp
from jax import lax
from jax.experimental import pallas as pl
from jax.experimental.pallas import tpu as pltpu
from jax.experimental.pallas import tpu_sc as plsc

BETA = 0.9
THRESHOLD = 1.0

N_NODES = 10000
N_EDGES = 320000
IN_CH = 129
OUT_CH = 64

NC = 2
NS = 16
NW = NC * NS
EPW = N_EDGES // NW
ROWS_MAIN = 624
ROWS_TAIL = N_NODES - NS * ROWS_MAIN
DEG_PAD = 10240

DEG_CHUNK = 2000
SC_CHUNK = 400

_MESH = plsc.VectorSubcoreMesh(core_axis_name="c", subcore_axis_name="s")
_SC_PARAMS = pltpu.CompilerParams(use_tc_tiling_on_sc=False)


@functools.partial(
    pl.kernel,
    out_type=jax.ShapeDtypeStruct((NC * DEG_PAD,), jnp.float32),
    mesh=_MESH,
    compiler_params=_SC_PARAMS,
    scratch_types=[
        pltpu.VMEM((DEG_CHUNK,), jnp.int32),
        pltpu.VMEM((DEG_CHUNK,), jnp.float32),
        pltpu.VMEM((DEG_PAD // NS,), jnp.float32),
        pltpu.VMEM_SHARED((DEG_PAD,), jnp.float32),
    ],
)
def _deg_kernel(dst_hbm, out_hbm, idx_v, ones_v, stage_v, deg_sp):
    c = lax.axis_index("c")
    s = lax.axis_index("s")
    wid = c * NS + s
    ones16 = jnp.ones((16,), jnp.float32)
    zeros16 = jnp.zeros((16,), jnp.float32)

    def _init(i, _):
        ones_v[pl.ds(i * 16, 16)] = ones16
        return 0

    lax.fori_loop(0, DEG_CHUNK // 16, _init, 0)

    def _zero(i, _):
        stage_v[pl.ds(i * 16, 16)] = zeros16
        return 0

    lax.fori_loop(0, (DEG_PAD // NS) // 16, _zero, 0)

    my = pl.ds(s * (DEG_PAD // NS), DEG_PAD // NS)
    pltpu.sync_copy(stage_v, deg_sp.at[my])
    plsc.subcore_barrier()

    base = wid * EPW

    def _chunk(j, _):
        pltpu.sync_copy(dst_hbm.at[pl.ds(base + j * DEG_CHUNK, DEG_CHUNK)], idx_v)
        pltpu.sync_copy(ones_v, deg_sp.at[idx_v], add=True)
        return 0

    lax.fori_loop(0, EPW // DEG_CHUNK, _chunk, 0)
    plsc.subcore_barrier()
    out_my = pl.ds(c * DEG_PAD + s * (DEG_PAD // NS), DEG_PAD // NS)
    pltpu.sync_copy(deg_sp.at[my], out_hbm.at[out_my])


@functools.partial(
    pl.kernel,
    out_type=jax.ShapeDtypeStruct((NC, N_NODES, OUT_CH), jnp.float32),
    mesh=_MESH,
    compiler_params=_SC_PARAMS,
    scratch_types=[
        pltpu.VMEM((SC_CHUNK,), jnp.int32),
        pltpu.VMEM((SC_CHUNK,), jnp.int32),
        pltpu.VMEM((SC_CHUNK, OUT_CH), jnp.float32),
        pltpu.VMEM_SHARED((N_NODES, OUT_CH), jnp.float32),
        pltpu.SemaphoreType.DMA,
    ],
)
def _msg_kernel(src_hbm, dst_hbm, g_hbm, out_hbm, sidx_v, didx_v, rows_v, acc_sp, sem):
    c = lax.axis_index("c")
    s = lax.axis_index("s")
    wid = c * NS + s
    myrows = pl.ds(s * ROWS_MAIN, ROWS_MAIN)
    tail = pl.ds(NS * ROWS_MAIN, ROWS_TAIL)

    pltpu.sync_copy(g_hbm.at[myrows], acc_sp.at[myrows])

    @pl.when(s == 0)
    def _seed_tail():
        pltpu.sync_copy(g_hbm.at[tail], acc_sp.at[tail])

    plsc.subcore_barrier()

    base = wid * EPW

    def _chunk(j, _):
        off = base + j * SC_CHUNK
        pltpu.sync_copy(src_hbm.at[pl.ds(off, SC_CHUNK)], sidx_v)
        pltpu.sync_copy(dst_hbm.at[pl.ds(off, SC_CHUNK)], didx_v)
        pltpu.async_copy(g_hbm.at[sidx_v], rows_v, sem).wait()
        pltpu.sync_copy(rows_v, acc_sp.at[didx_v], add=True)
        return 0

    lax.fori_loop(0, EPW // SC_CHUNK, _chunk, 0)
    plsc.subcore_barrier()
    pltpu.sync_copy(acc_sp.at[myrows], out_hbm.at[c, myrows])

    @pl.when(s == 0)
    def _write_tail():
        pltpu.sync_copy(acc_sp.at[tail], out_hbm.at[c, tail])


_TC_BLK = 1000


def _spk_h_body(x_ref, w_ref, spk_ref, h_ref):
    xb = x_ref[...]
    spk_ref[...] = (xb > THRESHOLD).astype(jnp.float32)
    h_ref[...] = jnp.dot(xb, w_ref[...], preferred_element_type=jnp.float32)


_spk_h = pl.pallas_call(
    _spk_h_body,
    grid=(N_NODES // _TC_BLK,),
    in_specs=[
        pl.BlockSpec((_TC_BLK, IN_CH), lambda i: (i, 0)),
        pl.BlockSpec((IN_CH, OUT_CH), lambda i: (0, 0)),
    ],
    out_specs=[
        pl.BlockSpec((_TC_BLK, IN_CH), lambda i: (i, 0)),
        pl.BlockSpec((_TC_BLK, OUT_CH), lambda i: (i, 0)),
    ],
    out_shape=[
        jax.ShapeDtypeStruct((N_NODES, IN_CH), jnp.float32),
        jax.ShapeDtypeStruct((N_NODES, OUT_CH), jnp.float32),
    ],
)


def _g_body(dpt_ref, h_ref, g_ref, dis_ref):
    deg = dpt_ref[..., 0:1] + dpt_ref[..., 1:2] + 1.0
    dis = lax.rsqrt(deg)
    dis_ref[...] = dis
    g_ref[...] = dis * h_ref[...]


_g_scale = pl.pallas_call(
    _g_body,
    grid=(N_NODES // _TC_BLK,),
    in_specs=[
        pl.BlockSpec((_TC_BLK, 2), lambda i: (i, 0)),
        pl.BlockSpec((_TC_BLK, OUT_CH), lambda i: (i, 0)),
    ],
    out_specs=[
        pl.BlockSpec((_TC_BLK, OUT_CH), lambda i: (i, 0)),
        pl.BlockSpec((_TC_BLK, 1), lambda i: (i, 0)),
    ],
    out_shape=[
        jax.ShapeDtypeStruct((N_NODES, OUT_CH), jnp.float32),
        jax.ShapeDtypeStruct((N_NODES, 1), jnp.float32),
    ],
)


def _comb_body(p0_ref, p1_ref, g_ref, dis_ref, b_ref, o_ref):
    o_ref[...] = (
        dis_ref[...] * (p0_ref[...] + p1_ref[...] - g_ref[...]) + b_ref[...]
    )


_combine = pl.pallas_call(
    _comb_body,
    grid=(N_NODES // _TC_BLK,),
    in_specs=[
        pl.BlockSpec((_TC_BLK, OUT_CH), lambda i: (i, 0)),
        pl.BlockSpec((_TC_BLK, OUT_CH), lambda i: (i, 0)),
        pl.BlockSpec((_TC_BLK, OUT_CH), lambda i: (i, 0)),
        pl.BlockSpec((_TC_BLK, 1), lambda i: (i, 0)),
        pl.BlockSpec((1, OUT_CH), lambda i: (0, 0)),
    ],
    out_specs=pl.BlockSpec((_TC_BLK, OUT_CH), lambda i: (i, 0)),
    out_shape=jax.ShapeDtypeStruct((N_NODES, OUT_CH), jnp.float32),
)


def kernel(x, edge_index, W, b):
    src = edge_index[0].astype(jnp.int32)
    dst = edge_index[1].astype(jnp.int32)

    spk, h = _spk_h(x, W)
    deg_p = _deg_kernel(dst).reshape(NC, DEG_PAD)
    dpt = deg_p[:, :N_NODES].T
    g, dis = _g_scale(dpt, h)
    pacc = _msg_kernel(src, dst, g)
    out = _combine(pacc[0], pacc[1], g, dis, b.reshape(1, OUT_CH))
    return (spk, out)


# speedup vs baseline: 53.1048x; 1.6077x over previous
import functools

import jax
import jax.numpy as jnp
from jax import lax
from jax.experimental import pallas as pl
from jax.experimental.pallas import tpu as pltpu
from jax.experimental.pallas import tpu_sc as plsc

BETA = 0.9
THRESHOLD = 1.0

N_NODES = 10000
N_EDGES = 320000
IN_CH = 129
OUT_CH = 64

NC = 2
NS = 16
NW = NC * NS
EPW = N_EDGES // NW
ROWS_MAIN = 624
ROWS_TAIL = N_NODES - NS * ROWS_MAIN
DEG_PAD = 10240

DEG_CHUNK = 2000
DEG_NCHUNK = EPW // DEG_CHUNK
SC_CHUNK = 200
SC_NCHUNK = EPW // SC_CHUNK
NBUF = 3

_MESH = plsc.VectorSubcoreMesh(core_axis_name="c", subcore_axis_name="s")
_SC_PARAMS = pltpu.CompilerParams(use_tc_tiling_on_sc=False)


@functools.partial(
    pl.kernel,
    out_type=jax.ShapeDtypeStruct((NC * DEG_PAD,), jnp.float32),
    mesh=_MESH,
    compiler_params=_SC_PARAMS,
    scratch_types=[
        pltpu.VMEM((DEG_NCHUNK, DEG_CHUNK), jnp.int32),
        pltpu.VMEM((DEG_CHUNK,), jnp.float32),
        pltpu.VMEM((DEG_PAD // NS,), jnp.float32),
        pltpu.VMEM_SHARED((DEG_PAD,), jnp.float32),
        pltpu.SemaphoreType.DMA,
        pltpu.SemaphoreType.DMA,
    ],
)
def _deg_kernel(ei_hbm, out_hbm, idx_v, ones_v, stage_v, deg_sp, isem, ssem):
    c = lax.axis_index("c")
    s = lax.axis_index("s")
    wid = c * NS + s
    base = wid * EPW
    ones16 = jnp.ones((16,), jnp.float32)
    zeros16 = jnp.zeros((16,), jnp.float32)

    idx_cps = [
        pltpu.async_copy(
            ei_hbm.at[1, pl.ds(base + j * DEG_CHUNK, DEG_CHUNK)],
            idx_v.at[j],
            isem,
        )
        for j in range(DEG_NCHUNK)
    ]

    def _init(i, _):
        ones_v[pl.ds(i * 16, 16)] = ones16
        return 0

    lax.fori_loop(0, DEG_CHUNK // 16, _init, 0)

    def _zero(i, _):
        stage_v[pl.ds(i * 16, 16)] = zeros16
        return 0

    lax.fori_loop(0, (DEG_PAD // NS) // 16, _zero, 0)

    my = pl.ds(s * (DEG_PAD // NS), DEG_PAD // NS)
    pltpu.sync_copy(stage_v, deg_sp.at[my])
    plsc.subcore_barrier()

    for cp in idx_cps:
        cp.wait()
    scat_cps = [
        pltpu.async_copy(ones_v, deg_sp.at[idx_v.at[j]], ssem, add=True)
        for j in range(DEG_NCHUNK)
    ]
    for cp in scat_cps:
        cp.wait()

    plsc.subcore_barrier()
    out_my = pl.ds(c * DEG_PAD + s * (DEG_PAD // NS), DEG_PAD // NS)
    pltpu.sync_copy(deg_sp.at[my], out_hbm.at[out_my])


@functools.partial(
    pl.kernel,
    out_type=[
        jax.ShapeDtypeStruct((N_NODES, OUT_CH), jnp.float32),
        jax.ShapeDtypeStruct((N_NODES, OUT_CH), jnp.float32),
    ],
    mesh=_MESH,
    compiler_params=_SC_PARAMS,
    scratch_types=[
        pltpu.VMEM((EPW,), jnp.int32),
        pltpu.VMEM((SC_NCHUNK, SC_CHUNK), jnp.int32),
        [pltpu.VMEM((SC_CHUNK, OUT_CH), jnp.float32) for _ in range(NBUF)],
        pltpu.VMEM_SHARED((N_NODES, OUT_CH), jnp.float32),
        pltpu.SemaphoreType.DMA,
        [pltpu.SemaphoreType.DMA for _ in range(NBUF)],
        [pltpu.SemaphoreType.DMA for _ in range(NBUF)],
    ],
)
def _msg_kernel(ei_hbm, g_hbm, out0_hbm, out1_hbm, sidx_v, didx_v, rows,
                acc_sp, isem, gsems, ssems):
    c = lax.axis_index("c")
    s = lax.axis_index("s")
    wid = c * NS + s
    base = wid * EPW
    myrows = pl.ds(s * ROWS_MAIN, ROWS_MAIN)
    tail = pl.ds(NS * ROWS_MAIN, ROWS_TAIL)

    src_cp = pltpu.async_copy(ei_hbm.at[0, pl.ds(base, EPW)], sidx_v, isem)
    didx_cps = [
        pltpu.async_copy(
            ei_hbm.at[1, pl.ds(base + j * SC_CHUNK, SC_CHUNK)],
            didx_v.at[j],
            isem,
        )
        for j in range(SC_NCHUNK)
    ]

    pltpu.sync_copy(g_hbm.at[myrows], acc_sp.at[myrows])

    @pl.when(s == 0)
    def _seed_tail():
        pltpu.sync_copy(g_hbm.at[tail], acc_sp.at[tail])

    src_cp.wait()
    for cp in didx_cps:
        cp.wait()

    def _gather(j):
        b = j % NBUF
        return pltpu.async_copy(
            g_hbm.at[sidx_v.at[pl.ds(j * SC_CHUNK, SC_CHUNK)]], rows[b],
            gsems[b],
        )

    gd = {0: _gather(0), 1: _gather(1)}
    sd = {}
    plsc.subcore_barrier()
    for j in range(SC_NCHUNK):
        nxt = j + 2
        if nxt < SC_NCHUNK:
            if nxt >= NBUF:
                sd[nxt - NBUF].wait()
            gd[nxt] = _gather(nxt)
        gd[j].wait()
        sd[j] = pltpu.async_copy(
            rows[j % NBUF], acc_sp.at[didx_v.at[j]], ssems[j % NBUF], add=True
        )
    for j in range(SC_NCHUNK - NBUF, SC_NCHUNK):
        sd[j].wait()

    plsc.subcore_barrier()
    myout = [out0_hbm, out1_hbm]
    for cc in range(NC):

        @pl.when(c == cc)
        def _writeback():
            pltpu.sync_copy(acc_sp.at[myrows], myout[cc].at[myrows])

            @pl.when(s == 0)
            def _write_tail():
                pltpu.sync_copy(acc_sp.at[tail], myout[cc].at[tail])


_TC_BLK = 1000


def _spk_g_body(x_ref, w_ref, dpt_ref, spk_ref, g_ref, dis_ref):
    xb = x_ref[...]
    spk_ref[...] = (xb > THRESHOLD).astype(jnp.float32)
    deg = dpt_ref[..., 0:1] + dpt_ref[..., 1:2] + 1.0
    dis = lax.rsqrt(deg)
    dis_ref[...] = dis
    g_ref[...] = dis * jnp.dot(
        xb, w_ref[...], preferred_element_type=jnp.float32
    )


_spk_g = pl.pallas_call(
    _spk_g_body,
    grid=(N_NODES // _TC_BLK,),
    in_specs=[
        pl.BlockSpec((_TC_BLK, IN_CH), lambda i: (i, 0)),
        pl.BlockSpec((IN_CH, OUT_CH), lambda i: (0, 0)),
        pl.BlockSpec((_TC_BLK, 2), lambda i: (i, 0)),
    ],
    out_specs=[
        pl.BlockSpec((_TC_BLK, IN_CH), lambda i: (i, 0)),
        pl.BlockSpec((_TC_BLK, OUT_CH), lambda i: (i, 0)),
        pl.BlockSpec((_TC_BLK, 1), lambda i: (i, 0)),
    ],
    out_shape=[
        jax.ShapeDtypeStruct((N_NODES, IN_CH), jnp.float32),
        jax.ShapeDtypeStruct((N_NODES, OUT_CH), jnp.float32),
        jax.ShapeDtypeStruct((N_NODES, 1), jnp.float32),
    ],
)


def _comb_body(p0_ref, p1_ref, g_ref, dis_ref, b_ref, o_ref):
    o_ref[...] = (
        dis_ref[...] * (p0_ref[...] + p1_ref[...] - g_ref[...]) + b_ref[...]
    )


_combine = pl.pallas_call(
    _comb_body,
    grid=(N_NODES // _TC_BLK,),
    in_specs=[
        pl.BlockSpec((_TC_BLK, OUT_CH), lambda i: (i, 0)),
        pl.BlockSpec((_TC_BLK, OUT_CH), lambda i: (i, 0)),
        pl.BlockSpec((_TC_BLK, OUT_CH), lambda i: (i, 0)),
        pl.BlockSpec((_TC_BLK, 1), lambda i: (i, 0)),
        pl.BlockSpec((1, OUT_CH), lambda i: (0, 0)),
    ],
    out_specs=pl.BlockSpec((_TC_BLK, OUT_CH), lambda i: (i, 0)),
    out_shape=jax.ShapeDtypeStruct((N_NODES, OUT_CH), jnp.float32),
)


def kernel(x, edge_index, W, b):
    ei = edge_index.astype(jnp.int32)
    deg_p = _deg_kernel(ei).reshape(NC, DEG_PAD)
    dpt = deg_p[:, :N_NODES].T
    spk, g, dis = _spk_g(x, W, dpt)
    p0, p1 = _msg_kernel(ei, g)
    out = _combine(p0, p1, g, dis, b.reshape(1, OUT_CH))
    return (spk, out)
